# baseline (speedup 1.0000x reference)
import jax
import jax.numpy as jnp
from jax.experimental import pallas as pl
from jax.experimental.pallas import tpu as pltpu

_B, _N, _D = 16, 2048, 3
_DP = 8


def _chamfer_body(p_ref, gt_ref, out_ref):
    b = pl.program_id(0)
    p = p_ref[0]
    g = gt_ref[0]

    p_sq = jnp.sum(p * p, axis=1, keepdims=True)
    g_sq = jnp.sum(g * g, axis=0, keepdims=True)

    n = p.shape[0]
    a = jnp.concatenate(
        [-2.0 * p[:, :_D], p_sq, jnp.ones((n, 1), jnp.float32),
         jnp.zeros((n, _DP - _D - 2), jnp.float32)], axis=1)
    bg = jnp.concatenate(
        [g[:_D, :], jnp.ones((1, n), jnp.float32), g_sq,
         jnp.zeros((_DP - _D - 2, n), jnp.float32)], axis=0)

    d = jnp.dot(a, bg, preferred_element_type=jnp.float32,
                precision=jax.lax.Precision.HIGHEST)

    s = jnp.sum(jnp.min(d, axis=1)) + jnp.sum(jnp.min(d, axis=0))

    @pl.when(b == 0)
    def _():
        out_ref[0, 0] = 0.0

    out_ref[0, 0] += s

    @pl.when(b == _B - 1)
    def _():
        out_ref[0, 0] = out_ref[0, 0] * (1.0 / (_B * _N))


def kernel(prediction, gt):
    p_pad = jnp.pad(prediction, ((0, 0), (0, 0), (0, _DP - _D)))
    g_t = jnp.pad(jnp.swapaxes(gt, 1, 2), ((0, 0), (0, _DP - _D), (0, 0)))

    out = pl.pallas_call(
        _chamfer_body,
        grid=(_B,),
        in_specs=[
            pl.BlockSpec((1, _N, _DP), lambda b: (b, 0, 0)),
            pl.BlockSpec((1, _DP, _N), lambda b: (b, 0, 0)),
        ],
        out_specs=pl.BlockSpec(memory_space=pltpu.SMEM),
        out_shape=jax.ShapeDtypeStruct((1, 1), jnp.float32),
        compiler_params=pltpu.CompilerParams(
            dimension_semantics=("arbitrary",),
        ),
    )(p_pad, g_t)
    return out[0, 0]


# speedup vs baseline: 1.7314x; 1.7314x over previous
import jax
import jax.numpy as jnp
from jax.experimental import pallas as pl
from jax.experimental.pallas import tpu as pltpu

_B, _N, _D = 16, 2048, 3
_DP = 8
_CHUNK = 32
_LANES = 128


def _chamfer_body(p_ref, gt_ref, out_ref, rowpart_ref):
    b = pl.program_id(0)
    p = p_ref[0]
    g = gt_ref[0]

    p_sq = jnp.sum(p * p, axis=1, keepdims=True)
    g_sq = jnp.sum(g * g, axis=0, keepdims=True)

    n = p.shape[0]
    a = jnp.concatenate(
        [-2.0 * p[:, :_D], jnp.ones((n, 1), jnp.float32),
         jnp.zeros((n, _DP - _D - 1), jnp.float32)], axis=1)
    bg = jnp.concatenate(
        [g[:_D, :], g_sq, jnp.zeros((_DP - _D - 1, n), jnp.float32)], axis=0)

    e = jnp.dot(a, bg, preferred_element_type=jnp.float32)

    acc = jnp.full((_CHUNK, n), jnp.inf, jnp.float32)
    for k in range(n // _CHUNK):
        ek = e[k * _CHUNK:(k + 1) * _CHUNK, :]
        psk = p_sq[k * _CHUNK:(k + 1) * _CHUNK, :]
        acc = jnp.minimum(acc, ek + psk)
        rowp = jnp.min(ek.reshape(_CHUNK, n // _LANES, _LANES), axis=1)
        rowpart_ref[k * _CHUNK:(k + 1) * _CHUNK, :] = rowp

    dist2 = jnp.min(acc, axis=0)
    dist1 = jnp.min(rowpart_ref[...], axis=1) + p_sq[:, 0]
    s = jnp.sum(dist1) + jnp.sum(dist2)

    @pl.when(b == 0)
    def _():
        out_ref[0, 0] = 0.0

    out_ref[0, 0] += s

    @pl.when(b == _B - 1)
    def _():
        out_ref[0, 0] = out_ref[0, 0] * (1.0 / (_B * _N))


def kernel(prediction, gt):
    p_pad = jnp.pad(prediction, ((0, 0), (0, 0), (0, _DP - _D)))
    g_t = jnp.pad(jnp.swapaxes(gt, 1, 2), ((0, 0), (0, _DP - _D), (0, 0)))

    out = pl.pallas_call(
        _chamfer_body,
        grid=(_B,),
        in_specs=[
            pl.BlockSpec((1, _N, _DP), lambda b: (b, 0, 0)),
            pl.BlockSpec((1, _DP, _N), lambda b: (b, 0, 0)),
        ],
        out_specs=pl.BlockSpec(memory_space=pltpu.SMEM),
        out_shape=jax.ShapeDtypeStruct((1, 1), jnp.float32),
        scratch_shapes=[pltpu.VMEM((_N, _LANES), jnp.float32)],
        compiler_params=pltpu.CompilerParams(
            dimension_semantics=("arbitrary",),
        ),
    )(p_pad, g_t)
    return out[0, 0]


# speedup vs baseline: 3.5147x; 2.0300x over previous
import jax
import jax.numpy as jnp
from jax.experimental import pallas as pl
from jax.experimental.pallas import tpu as pltpu

_B, _N, _D = 16, 2048, 3
_DP = 8
_CHUNK = 32
_LANES = 128


def _chamfer_body(p_ref, gt_ref, out_ref):
    b = pl.program_id(0)
    p = p_ref[0]
    g = gt_ref[0]

    p_sq = jnp.sum(p * p, axis=1, keepdims=True)
    g_sq = jnp.sum(g * g, axis=0, keepdims=True)

    n = p.shape[0]
    a = jnp.concatenate(
        [-2.0 * p[:, :_D], jnp.ones((n, 1), jnp.float32),
         jnp.zeros((n, _DP - _D - 1), jnp.float32)], axis=1)
    bg = jnp.concatenate(
        [g[:_D, :], g_sq, jnp.zeros((_DP - _D - 1, n), jnp.float32)], axis=0)

    e = jnp.dot(a, bg, preferred_element_type=jnp.float32)

    d = e + p_sq
    s = jnp.sum(jnp.min(d, axis=1)) + jnp.sum(jnp.min(d, axis=0))

    @pl.when(b == 0)
    def _():
        out_ref[0, 0] = 0.0

    out_ref[0, 0] += s

    @pl.when(b == _B - 1)
    def _():
        out_ref[0, 0] = out_ref[0, 0] * (1.0 / (_B * _N))


def kernel(prediction, gt):
    p_pad = jnp.pad(prediction, ((0, 0), (0, 0), (0, _DP - _D)))
    g_t = jnp.pad(jnp.swapaxes(gt, 1, 2), ((0, 0), (0, _DP - _D), (0, 0)))

    out = pl.pallas_call(
        _chamfer_body,
        grid=(_B,),
        in_specs=[
            pl.BlockSpec((1, _N, _DP), lambda b: (b, 0, 0)),
            pl.BlockSpec((1, _DP, _N), lambda b: (b, 0, 0)),
        ],
        out_specs=pl.BlockSpec(memory_space=pltpu.SMEM),
        out_shape=jax.ShapeDtypeStruct((1, 1), jnp.float32),
        compiler_params=pltpu.CompilerParams(
            dimension_semantics=("arbitrary",),
        ),
    )(p_pad, g_t)
    return out[0, 0]


# speedup vs baseline: 3.6186x; 1.0296x over previous
import jax
import jax.numpy as jnp
from jax.experimental import pallas as pl
from jax.experimental.pallas import tpu as pltpu

_B, _N, _D = 16, 2048, 3
_DP = 8
_BB = 4


def _one_batch(p, g):
    n = p.shape[0]
    p_sq = jnp.sum(p * p, axis=1, keepdims=True)
    g_sq = jnp.sum(g * g, axis=0, keepdims=True)

    a = jnp.concatenate(
        [-2.0 * p[:, :_D], jnp.ones((n, 1), jnp.float32),
         jnp.zeros((n, _DP - _D - 1), jnp.float32)], axis=1)
    bg = jnp.concatenate(
        [g[:_D, :], g_sq, jnp.zeros((_DP - _D - 1, n), jnp.float32)], axis=0)

    e = jnp.dot(a, bg, preferred_element_type=jnp.float32)
    d = e + p_sq
    return jnp.sum(jnp.min(d, axis=1)) + jnp.sum(jnp.min(d, axis=0))


def _chamfer_body(p_ref, gt_ref, out_ref):
    step = pl.program_id(0)

    s = 0.0
    for i in range(_BB):
        s += _one_batch(p_ref[i], gt_ref[i])

    @pl.when(step == 0)
    def _():
        out_ref[0, 0] = 0.0

    out_ref[0, 0] += s

    @pl.when(step == _B // _BB - 1)
    def _():
        out_ref[0, 0] = out_ref[0, 0] * (1.0 / (_B * _N))


def kernel(prediction, gt):
    p_pad = jnp.pad(prediction, ((0, 0), (0, 0), (0, _DP - _D)))
    g_t = jnp.pad(jnp.swapaxes(gt, 1, 2), ((0, 0), (0, _DP - _D), (0, 0)))

    out = pl.pallas_call(
        _chamfer_body,
        grid=(_B // _BB,),
        in_specs=[
            pl.BlockSpec((_BB, _N, _DP), lambda b: (b, 0, 0)),
            pl.BlockSpec((_BB, _DP, _N), lambda b: (b, 0, 0)),
        ],
        out_specs=pl.BlockSpec(memory_space=pltpu.SMEM),
        out_shape=jax.ShapeDtypeStruct((1, 1), jnp.float32),
        compiler_params=pltpu.CompilerParams(
            dimension_semantics=("arbitrary",),
        ),
    )(p_pad, g_t)
    return out[0, 0]


# speedup vs baseline: 3.7485x; 1.0359x over previous
import jax
import jax.numpy as jnp
from jax.experimental import pallas as pl
from jax.experimental.pallas import tpu as pltpu

_B, _N, _D = 16, 2048, 3

_DIMS = (((1,), (1,)), ((), ()))


def _chamfer_body(p_ref, gt_ref, out_ref):
    b = pl.program_id(0)
    p = p_ref[0]
    g = gt_ref[0]

    p_sq = jnp.sum(p * p, axis=1, keepdims=True)
    g_sq = jnp.sum(g * g, axis=1, keepdims=True)

    n = p.shape[0]
    a = jnp.concatenate([-2.0 * p, jnp.ones((n, 1), jnp.float32)], axis=1)
    gg = jnp.concatenate([g, g_sq], axis=1)

    e = jax.lax.dot_general(a, gg, _DIMS,
                            preferred_element_type=jnp.float32)
    d = e + p_sq
    s = jnp.sum(jnp.min(d, axis=1)) + jnp.sum(jnp.min(d, axis=0))

    @pl.when(b == 0)
    def _():
        out_ref[0, 0] = 0.0

    out_ref[0, 0] += s

    @pl.when(b == _B - 1)
    def _():
        out_ref[0, 0] = out_ref[0, 0] * (1.0 / (_B * _N))


def kernel(prediction, gt):
    out = pl.pallas_call(
        _chamfer_body,
        grid=(_B,),
        in_specs=[
            pl.BlockSpec((1, _N, _D), lambda b: (b, 0, 0)),
            pl.BlockSpec((1, _N, _D), lambda b: (b, 0, 0)),
        ],
        out_specs=pl.BlockSpec(memory_space=pltpu.SMEM),
        out_shape=jax.ShapeDtypeStruct((1, 1), jnp.float32),
        compiler_params=pltpu.CompilerParams(
            dimension_semantics=("arbitrary",),
        ),
    )(prediction, gt)
    return out[0, 0]


# speedup vs baseline: 4.6676x; 1.2452x over previous
import jax
import jax.numpy as jnp
from jax.experimental import pallas as pl
from jax.experimental.pallas import tpu as pltpu

_B, _N, _D = 16, 2048, 3

_DIMS = (((0,), (0,)), ((), ()))


def _chamfer_body(p_ref, gt_ref, out_ref):
    b = pl.program_id(0)
    p = p_ref[0]
    g = gt_ref[0]

    n = p.shape[1]
    p_sq = jnp.sum(p * p, axis=0, keepdims=True)
    g_sq = jnp.sum(g * g, axis=0, keepdims=True)

    a = jnp.concatenate([-2.0 * p, jnp.ones((1, n), jnp.float32)], axis=0)
    gg = jnp.concatenate([g, g_sq], axis=0)

    e = jax.lax.dot_general(a, gg, _DIMS,
                            preferred_element_type=jnp.float32)
    rowmin = jnp.min(e, axis=1)
    s1 = jnp.sum(rowmin) + jnp.sum(p_sq)
    s2 = jnp.sum(jnp.min(e + p_sq.T, axis=0))

    s = s1 + s2

    @pl.when(b == 0)
    def _():
        out_ref[0, 0] = 0.0

    out_ref[0, 0] += s

    @pl.when(b == _B - 1)
    def _():
        out_ref[0, 0] = out_ref[0, 0] * (1.0 / (_B * _N))


def kernel(prediction, gt):
    p_t = jnp.swapaxes(prediction, 1, 2)
    g_t = jnp.swapaxes(gt, 1, 2)

    out = pl.pallas_call(
        _chamfer_body,
        grid=(_B,),
        in_specs=[
            pl.BlockSpec((1, _D, _N), lambda b: (b, 0, 0)),
            pl.BlockSpec((1, _D, _N), lambda b: (b, 0, 0)),
        ],
        out_specs=pl.BlockSpec(memory_space=pltpu.SMEM),
        out_shape=jax.ShapeDtypeStruct((1, 1), jnp.float32),
        compiler_params=pltpu.CompilerParams(
            dimension_semantics=("arbitrary",),
        ),
    )(p_t, g_t)
    return out[0, 0]


# speedup vs baseline: 5.1219x; 1.0973x over previous
import jax
import jax.numpy as jnp
from jax.experimental import pallas as pl
from jax.experimental.pallas import tpu as pltpu

_B, _N, _D = 16, 2048, 3

_DIMS = (((0,), (0,)), ((), ()))


def _split3(x):
    hi = x.astype(jnp.bfloat16)
    r = x - hi.astype(jnp.float32)
    mid = r.astype(jnp.bfloat16)
    lo = (r - mid.astype(jnp.float32)).astype(jnp.bfloat16)
    return hi, mid, lo


def _vsum(v):
    v2 = v.reshape(16, 128)
    return jnp.sum(jnp.sum(v2, axis=1))


def _chamfer_body(p_ref, gt_ref, out_ref):
    b = pl.program_id(0)
    p = p_ref[0]
    g = gt_ref[0]
    n = p.shape[1]

    p_sq = jnp.sum(p * p, axis=0, keepdims=True)
    g_sq = jnp.sum(g * g, axis=0, keepdims=True)

    ah = (-2.0 * p).astype(jnp.bfloat16)
    gh = g.astype(jnp.bfloat16)
    ps_h, ps_m, ps_l = _split3(p_sq)
    gs_h, gs_m, gs_l = _split3(g_sq)
    one = jnp.ones((1, n), jnp.bfloat16)

    a = jnp.concatenate(
        [ah, ps_h, ps_m, ps_l, one, one, one], axis=0)
    gg = jnp.concatenate(
        [gh, one, one, one, gs_h, gs_m, gs_l], axis=0)

    d = jax.lax.dot_general(a, gg, _DIMS,
                            preferred_element_type=jnp.float32)

    s = _vsum(jnp.min(d, axis=1)) + _vsum(jnp.min(d, axis=0))
    s = s * (1.0 / (_B * _N))

    @pl.when(b == 0)
    def _():
        out_ref[0, 0] = 0.0

    out_ref[0, 0] += s


def kernel(prediction, gt):
    p_t = jnp.swapaxes(prediction, 1, 2)
    g_t = jnp.swapaxes(gt, 1, 2)

    out = pl.pallas_call(
        _chamfer_body,
        grid=(_B,),
        in_specs=[
            pl.BlockSpec((1, _D, _N), lambda b: (b, 0, 0)),
            pl.BlockSpec((1, _D, _N), lambda b: (b, 0, 0)),
        ],
        out_specs=pl.BlockSpec(memory_space=pltpu.SMEM),
        out_shape=jax.ShapeDtypeStruct((1, 1), jnp.float32),
        compiler_params=pltpu.CompilerParams(
            dimension_semantics=("arbitrary",),
        ),
    )(p_t, g_t)
    return out[0, 0]
